# baseline (speedup 1.0000x reference)
import functools

import jax
import jax.numpy as jnp
from jax import lax
from jax.experimental import pallas as pl
from jax.experimental.pallas import tpu as pltpu
from jax.experimental.pallas import tpu_sc as plsc

N = 2048
NPROJ = 100
BS = 16
L = 16
M = N // L
NB = 64
NPASS = 6
TPW = NPROJ // 2
MININT = -2147483648


def _prep_body(x_ref, p_ref, proj_ref, w_ref):
    x = x_ref[0, 0]
    p = p_ref[...]
    x0 = x[0:1, :]
    x1 = x[1:2, :]
    x2 = x[2:3, :]
    proj_ref[0, 0] = p[:, 0:1] * x0 + p[:, 1:2] * x1 + p[:, 2:3] * x2
    s = (x1 + x0) ** 2
    w = s / jnp.sum(s + 1e-8)
    w_ref[0, 0] = jnp.broadcast_to(w, (8, N))


_prep = pl.pallas_call(
    _prep_body,
    grid=(2, BS),
    in_specs=[
        pl.BlockSpec((1, 1, 3, N), lambda g, s: (g, s, 0, 0)),
        pl.BlockSpec((NPROJ, 3), lambda g, s: (0, 0)),
    ],
    out_specs=[
        pl.BlockSpec((1, 1, NPROJ, N), lambda g, s: (g, s, 0, 0)),
        pl.BlockSpec((1, 1, 8, N), lambda g, s: (g, s, 0, 0)),
    ],
    out_shape=[
        jax.ShapeDtypeStruct((2, BS, NPROJ, N), jnp.float32),
        jax.ShapeDtypeStruct((2, BS, 8, N), jnp.float32),
    ],
)


def _shr_l(x, n):
    return lax.shift_right_logical(x, jnp.full((L,), n, jnp.int32))


def _shr_a(x, n):
    return lax.shift_right_arithmetic(x, jnp.full((L,), n, jnp.int32))


def _build_keys(raw, w_loc, kA, pA):
    def body(i, c):
        x = raw[pl.ds(i * L, L)]
        k = lax.bitcast_convert_type(x, jnp.int32)
        m = _shr_a(k, 31)
        kA[pl.ds(i * L, L)] = k ^ (m | MININT)
        pA[pl.ds(i * L, L)] = w_loc[pl.ds(i * L, L)]
        return c
    lax.fori_loop(0, M, body, 0, unroll=4)


def _radix_pass(src_k, src_p, dst_k, dst_p, shift, last, hist, cnt, lanes):
    zero16 = jnp.zeros((L,), jnp.int32)
    ones16 = jnp.ones((L,), jnp.int32)

    def zero_body(i, c):
        hist[pl.ds(i * L, L)] = zero16
        return c
    lax.fori_loop(0, NB, zero_body, 0, unroll=8)

    def hist_body(i, c):
        kv = src_k[pl.ds(i * L, L)]
        dig = _shr_l(kv, shift) & (NB - 1)
        idx = dig * L + lanes
        hv = plsc.load_gather(hist, [idx])
        plsc.store_scatter(hist, [idx], hv + ones16)
        return c
    lax.fori_loop(0, M, hist_body, 0, unroll=4)

    def scan_body(d, carry):
        hv = hist[pl.ds(d * L, L)]
        csum = jnp.cumsum(hv)
        cnt[pl.ds(d * L, L)] = csum - hv + carry
        return carry + jnp.sum(hv)
    lax.fori_loop(0, NB, scan_body, jnp.int32(0), unroll=4)

    def perm_body(i, c):
        kv = src_k[pl.ds(i * L, L)]
        pv = src_p[pl.ds(i * L, L)]
        idx = (_shr_l(kv, shift) & (NB - 1)) * L + lanes
        q = plsc.load_gather(cnt, [idx])
        plsc.store_scatter(cnt, [idx], q + 1)
        if last:
            addr = q
        else:
            addr = (q & (M - 1)) * L + _shr_l(q, 7)
        plsc.store_scatter(dst_k, [addr], kv)
        plsc.store_scatter(dst_p, [addr], pv)
        return c
    lax.fori_loop(0, M, perm_body, 0, unroll=4)


def _sort(kA, pA, kB, pB, hist, cnt, lanes):
    for pno in range(NPASS):
        if pno % 2 == 0:
            _radix_pass(kA, pA, kB, pB, 6 * pno, pno == NPASS - 1, hist, cnt, lanes)
        else:
            _radix_pass(kB, pB, kA, pA, 6 * pno, pno == NPASS - 1, hist, cnt, lanes)


def _finish(kA, pA, vals_out, cumpad):
    cumpad[pl.ds(0, L)] = jnp.zeros((L,), jnp.float32)

    def body(i, carry):
        y = kA[pl.ds(i * L, L)]
        m = _shr_a(~y, 31)
        vals_out[pl.ds(i * L, L)] = lax.bitcast_convert_type(y ^ (m | MININT), jnp.float32)
        av = pA[pl.ds(i * L, L)]
        cumpad[pl.ds(i * L + 1, L)] = jnp.cumsum(av) + carry
        return carry + jnp.sum(av)
    return lax.fori_loop(0, M, body, jnp.float32(0.0), unroll=4)


def _clamp_last(cumpad, t_val, lanes):
    v = cumpad[pl.ds(N - L + 1, L)]
    cumpad[pl.ds(N - L + 1, L)] = jnp.where(lanes == L - 1, t_val, v)


def _build_c(vs, v_pad, c_pad):
    c_pad[pl.ds(0, L)] = jnp.zeros((L,), jnp.float32)

    def body(i, carry):
        vv = vs[pl.ds(i * L, L)]
        dv = v_pad[pl.ds(i * L + 1, L)] - v_pad[pl.ds(i * L, L)]
        pr = vv * dv
        c_pad[pl.ds(i * L + 1, L)] = jnp.cumsum(pr) + carry
        return carry + jnp.sum(pr)
    lax.fori_loop(0, M, body, jnp.float32(0.0), unroll=4)


def _search(u_pad, v_pad, c_pad, vs, g_pad):
    g_pad[pl.ds(0, L)] = jnp.zeros((L,), jnp.float32)
    zero_i = jnp.zeros((L,), jnp.int32)
    n_i = jnp.full((L,), N, jnp.int32)
    one_i = jnp.full((L,), 1, jnp.int32)

    def body(i, c):
        q = u_pad[pl.ds(i * L + 1, L)]

        def bs(s, lh):
            lo, hi = lh
            mid = lax.shift_right_logical(lo + hi, one_i)
            vm = plsc.load_gather(v_pad, [mid + 1])
            go_right = vm < q
            return (jnp.where(go_right, mid + 1, lo),
                    jnp.where(go_right, hi, mid))
        lo, _ = lax.fori_loop(0, 12, bs, (zero_i, n_i))
        j = jnp.minimum(lo, N - 1)
        c_prev = plsc.load_gather(c_pad, [j])
        v_prev = plsc.load_gather(v_pad, [j])
        vj = plsc.load_gather(vs, [j])
        g_pad[pl.ds(i * L + 1, L)] = c_prev + vj * (q - v_prev)
        return c
    lax.fori_loop(0, M, body, 0, unroll=2)


def _accum(us, vs, u_pad, v_pad, g_pad):
    def body(i, acc):
        uu = us[pl.ds(i * L, L)]
        vv = vs[pl.ds(i * L, L)]
        du = u_pad[pl.ds(i * L + 1, L)] - u_pad[pl.ds(i * L, L)]
        dv = v_pad[pl.ds(i * L + 1, L)] - v_pad[pl.ds(i * L, L)]
        dg = g_pad[pl.ds(i * L + 1, L)] - g_pad[pl.ds(i * L, L)]
        return acc + uu * uu * du + vv * vv * dv - 2.0 * uu * dg
    return lax.fori_loop(0, M, body, jnp.zeros((L,), jnp.float32), unroll=4)


@functools.lru_cache(maxsize=1)
def _make_sw_kernel():
    mesh = plsc.VectorSubcoreMesh(core_axis_name="c", subcore_axis_name="s")
    return functools.partial(
        pl.kernel,
        mesh=mesh,
        compiler_params=pltpu.CompilerParams(needs_layout_passes=False),
        out_type=jax.ShapeDtypeStruct((BS * NPROJ * L,), jnp.float32),
        scratch_types=[
        pltpu.VMEM((N,), jnp.float32),
        pltpu.VMEM((N,), jnp.int32),
        pltpu.VMEM((N,), jnp.float32),
        pltpu.VMEM((N,), jnp.int32),
        pltpu.VMEM((N,), jnp.float32),
        pltpu.VMEM((N,), jnp.float32),
        pltpu.VMEM((N,), jnp.float32),
        pltpu.VMEM((N + L,), jnp.float32),
        pltpu.VMEM((N + L,), jnp.float32),
        pltpu.VMEM((N + L,), jnp.float32),
        pltpu.VMEM((N + L,), jnp.float32),
        pltpu.VMEM((NB * L,), jnp.int32),
        pltpu.VMEM((NB * L,), jnp.int32),
        pltpu.VMEM((N,), jnp.float32),
        pltpu.VMEM((N,), jnp.float32),
        pltpu.VMEM((TPW * L,), jnp.float32),
        ],
    )(_sw_body)


def _sw_body(up_hbm, vp_hbm, a_hbm, b_hbm, out_hbm, raw, kA, pA, kB, pB,
             us, vs, u_pad, v_pad, c_pad, g_pad, hist, cnt,
             a_loc, b_loc, res):
    s_idx = lax.axis_index("s")
    half = lax.axis_index("c")
    t0 = s_idx * NPROJ + half * TPW
    lanes = lax.iota(jnp.int32, L)

    pltpu.sync_copy(a_hbm.at[s_idx * 8], a_loc)
    pltpu.sync_copy(b_hbm.at[s_idx * 8], b_loc)

    def task(k, c):
        t = t0 + k
        pltpu.sync_copy(up_hbm.at[t], raw)
        _build_keys(raw, a_loc, kA, pA)
        _sort(kA, pA, kB, pB, hist, cnt, lanes)
        tot_u = _finish(kA, pA, us, u_pad)

        pltpu.sync_copy(vp_hbm.at[t], raw)
        _build_keys(raw, b_loc, kA, pA)
        _sort(kA, pA, kB, pB, hist, cnt, lanes)
        tot_v = _finish(kA, pA, vs, v_pad)

        t_val = jnp.maximum(tot_u, tot_v)
        _clamp_last(u_pad, t_val, lanes)
        _clamp_last(v_pad, t_val, lanes)
        _build_c(vs, v_pad, c_pad)
        _search(u_pad, v_pad, c_pad, vs, g_pad)
        res[pl.ds(k * L, L)] = _accum(us, vs, u_pad, v_pad, g_pad)
        return c
    lax.fori_loop(0, TPW, task, 0)
    pltpu.sync_copy(res, out_hbm.at[pl.ds(t0 * L, TPW * L)])


def kernel(set1, set2, projections):
    xt = jnp.stack([set1.transpose(0, 2, 1), set2.transpose(0, 2, 1)])
    proj, w = _prep(xt, projections.T)
    up = proj[0].reshape(BS * NPROJ, N)
    vp = proj[1].reshape(BS * NPROJ, N)
    a2 = w[0].reshape(BS * 8, N)
    b2 = w[1].reshape(BS * 8, N)
    out = _make_sw_kernel()(up, vp, a2, b2)
    per_task = out.reshape(BS, NPROJ, L).sum(-1)
    return jnp.sum(jnp.mean(per_task, axis=1) ** 0.5)


# speedup vs baseline: 1.7460x; 1.7460x over previous
import functools

import jax
import jax.numpy as jnp
from jax import lax
from jax.experimental import pallas as pl
from jax.experimental.pallas import tpu as pltpu
from jax.experimental.pallas import tpu_sc as plsc

N = 2048
NPROJ = 100
BS = 16
L = 16
M = N // L
NB = 64
NPASS = 6
TPW = NPROJ // 2
MININT = -2147483648


def _prep_body(x_ref, p_ref, proj_ref, w_ref):
    x = x_ref[0, 0]
    p = p_ref[...]
    x0 = x[0:1, :]
    x1 = x[1:2, :]
    x2 = x[2:3, :]
    proj_ref[0, 0] = p[:, 0:1] * x0 + p[:, 1:2] * x1 + p[:, 2:3] * x2
    s = (x1 + x0) ** 2
    w = s / jnp.sum(s + 1e-8)
    w_ref[0, 0] = jnp.broadcast_to(w, (8, N))


_prep = pl.pallas_call(
    _prep_body,
    grid=(2, BS),
    in_specs=[
        pl.BlockSpec((1, 1, 3, N), lambda g, s: (g, s, 0, 0)),
        pl.BlockSpec((NPROJ, 3), lambda g, s: (0, 0)),
    ],
    out_specs=[
        pl.BlockSpec((1, 1, NPROJ, N), lambda g, s: (g, s, 0, 0)),
        pl.BlockSpec((1, 1, 8, N), lambda g, s: (g, s, 0, 0)),
    ],
    out_shape=[
        jax.ShapeDtypeStruct((2, BS, NPROJ, N), jnp.float32),
        jax.ShapeDtypeStruct((2, BS, 8, N), jnp.float32),
    ],
)


def _shr_l(x, n):
    return lax.shift_right_logical(x, jnp.full((L,), n, jnp.int32))


def _shr_a(x, n):
    return lax.shift_right_arithmetic(x, jnp.full((L,), n, jnp.int32))


def _build_keys2(raw_u, raw_v, a_loc, b_loc, kA, pA, kC, pC):
    def body(i, c):
        sl = pl.ds(i * L, L)
        ku = lax.bitcast_convert_type(raw_u[sl], jnp.int32)
        kv = lax.bitcast_convert_type(raw_v[sl], jnp.int32)
        kA[sl] = ku ^ (_shr_a(ku, 31) | MININT)
        kC[sl] = kv ^ (_shr_a(kv, 31) | MININT)
        pA[sl] = a_loc[sl]
        pC[sl] = b_loc[sl]
        return c
    lax.fori_loop(0, M, body, 0)


def _radix_pass2(bufs, shift, last, hist_u, cnt_u, hist_v, cnt_v, lanes):
    (src_ku, src_pu, dst_ku, dst_pu, src_kv, src_pv, dst_kv, dst_pv) = bufs
    zero16 = jnp.zeros((L,), jnp.int32)
    ones16 = jnp.ones((L,), jnp.int32)

    def zero_body(i, c):
        sl = pl.ds(i * L, L)
        hist_u[sl] = zero16
        hist_v[sl] = zero16
        return c
    lax.fori_loop(0, NB, zero_body, 0)

    def hist_body(i, c):
        sl = pl.ds(i * L, L)
        iu = (_shr_l(src_ku[sl], shift) & (NB - 1)) * L + lanes
        iv = (_shr_l(src_kv[sl], shift) & (NB - 1)) * L + lanes
        hu = plsc.load_gather(hist_u, [iu])
        hv = plsc.load_gather(hist_v, [iv])
        plsc.store_scatter(hist_u, [iu], hu + ones16)
        plsc.store_scatter(hist_v, [iv], hv + ones16)
        return c
    lax.fori_loop(0, M, hist_body, 0)

    def scan_body(d, carry):
        cu, cv = carry
        sl = pl.ds(d * L, L)
        hu = hist_u[sl]
        hv = hist_v[sl]
        csu = jnp.cumsum(hu)
        csv = jnp.cumsum(hv)
        cnt_u[sl] = csu - hu + cu
        cnt_v[sl] = csv - hv + cv
        return (cu + csu[L - 1], cv + csv[L - 1])
    lax.fori_loop(0, NB, scan_body, (jnp.int32(0), jnp.int32(0)))

    def perm_body(i, c):
        sl = pl.ds(i * L, L)
        ku = src_ku[sl]
        pu = src_pu[sl]
        kv = src_kv[sl]
        pv = src_pv[sl]
        iu = (_shr_l(ku, shift) & (NB - 1)) * L + lanes
        iv = (_shr_l(kv, shift) & (NB - 1)) * L + lanes
        qu = plsc.load_gather(cnt_u, [iu])
        qv = plsc.load_gather(cnt_v, [iv])
        plsc.store_scatter(cnt_u, [iu], qu + 1)
        plsc.store_scatter(cnt_v, [iv], qv + 1)
        if last:
            au, av = qu, qv
        else:
            au = (qu & (M - 1)) * L + _shr_l(qu, 7)
            av = (qv & (M - 1)) * L + _shr_l(qv, 7)
        plsc.store_scatter(dst_ku, [au], ku)
        plsc.store_scatter(dst_pu, [au], pu)
        plsc.store_scatter(dst_kv, [av], kv)
        plsc.store_scatter(dst_pv, [av], pv)
        return c
    lax.fori_loop(0, M, perm_body, 0)


def _sort2(kA, pA, kB, pB, kC, pC, kD, pD, hist_u, cnt_u, hist_v, cnt_v, lanes):
    for pno in range(NPASS):
        if pno % 2 == 0:
            bufs = (kA, pA, kB, pB, kC, pC, kD, pD)
        else:
            bufs = (kB, pB, kA, pA, kD, pD, kC, pC)
        _radix_pass2(bufs, 6 * pno, pno == NPASS - 1,
                     hist_u, cnt_u, hist_v, cnt_v, lanes)


def _finish2(kA, pA, kC, pC, us, vs, u_pad, v_pad):
    zf = jnp.zeros((L,), jnp.float32)
    u_pad[pl.ds(0, L)] = zf
    v_pad[pl.ds(0, L)] = zf

    def body(i, carry):
        cu, cv = carry
        sl = pl.ds(i * L, L)
        sl1 = pl.ds(i * L + 1, L)
        yu = kA[sl]
        yv = kC[sl]
        us[sl] = lax.bitcast_convert_type(yu ^ (_shr_a(~yu, 31) | MININT), jnp.float32)
        vs[sl] = lax.bitcast_convert_type(yv ^ (_shr_a(~yv, 31) | MININT), jnp.float32)
        au = pA[sl]
        av = pC[sl]
        csu = jnp.cumsum(au) + cu
        csv = jnp.cumsum(av) + cv
        u_pad[sl1] = csu
        v_pad[sl1] = csv
        return (csu[L - 1], csv[L - 1])
    return lax.fori_loop(0, M, body, (jnp.float32(0.0), jnp.float32(0.0)))


def _clamp_last(cumpad, t_val, lanes):
    v = cumpad[pl.ds(N - L + 1, L)]
    cumpad[pl.ds(N - L + 1, L)] = jnp.where(lanes == L - 1, t_val, v)


def _build_c(vs, v_pad, c_pad):
    c_pad[pl.ds(0, L)] = jnp.zeros((L,), jnp.float32)

    def body(i, carry):
        vv = vs[pl.ds(i * L, L)]
        dv = v_pad[pl.ds(i * L + 1, L)] - v_pad[pl.ds(i * L, L)]
        pr = vv * dv
        cs = jnp.cumsum(pr) + carry
        c_pad[pl.ds(i * L + 1, L)] = cs
        return cs[L - 1]
    lax.fori_loop(0, M, body, jnp.float32(0.0))


def _search(u_pad, v_pad, c_pad, vs, g_pad):
    g_pad[pl.ds(0, L)] = jnp.zeros((L,), jnp.float32)
    zero_i = jnp.zeros((L,), jnp.int32)
    n_i = jnp.full((L,), N, jnp.int32)
    one_i = jnp.full((L,), 1, jnp.int32)

    def body(i, c):
        q1 = u_pad[pl.ds(2 * i * L + 1, L)]
        q2 = u_pad[pl.ds((2 * i + 1) * L + 1, L)]

        def bs(s, lh):
            lo1, hi1, lo2, hi2 = lh
            m1 = lax.shift_right_logical(lo1 + hi1, one_i)
            m2 = lax.shift_right_logical(lo2 + hi2, one_i)
            f1 = plsc.load_gather(v_pad, [m1 + 1])
            f2 = plsc.load_gather(v_pad, [m2 + 1])
            g1 = f1 < q1
            g2 = f2 < q2
            return (jnp.where(g1, m1 + 1, lo1), jnp.where(g1, hi1, m1),
                    jnp.where(g2, m2 + 1, lo2), jnp.where(g2, hi2, m2))
        lo1, _, lo2, _ = lax.fori_loop(0, 12, bs, (zero_i, n_i, zero_i, n_i))
        j1 = jnp.minimum(lo1, N - 1)
        j2 = jnp.minimum(lo2, N - 1)
        cp1 = plsc.load_gather(c_pad, [j1])
        cp2 = plsc.load_gather(c_pad, [j2])
        vp1 = plsc.load_gather(v_pad, [j1])
        vp2 = plsc.load_gather(v_pad, [j2])
        vj1 = plsc.load_gather(vs, [j1])
        vj2 = plsc.load_gather(vs, [j2])
        g_pad[pl.ds(2 * i * L + 1, L)] = cp1 + vj1 * (q1 - vp1)
        g_pad[pl.ds((2 * i + 1) * L + 1, L)] = cp2 + vj2 * (q2 - vp2)
        return c
    lax.fori_loop(0, M // 2, body, 0)


def _accum(us, vs, u_pad, v_pad, g_pad):
    def body(i, acc):
        a1, a2 = acc
        s1 = pl.ds(2 * i * L, L)
        s2 = pl.ds((2 * i + 1) * L, L)
        s1p = pl.ds(2 * i * L + 1, L)
        s2p = pl.ds((2 * i + 1) * L + 1, L)
        uu1 = us[s1]
        uu2 = us[s2]
        vv1 = vs[s1]
        vv2 = vs[s2]
        du1 = u_pad[s1p] - u_pad[s1]
        du2 = u_pad[s2p] - u_pad[s2]
        dv1 = v_pad[s1p] - v_pad[s1]
        dv2 = v_pad[s2p] - v_pad[s2]
        dg1 = g_pad[s1p] - g_pad[s1]
        dg2 = g_pad[s2p] - g_pad[s2]
        a1 = a1 + uu1 * uu1 * du1 + vv1 * vv1 * dv1 - 2.0 * uu1 * dg1
        a2 = a2 + uu2 * uu2 * du2 + vv2 * vv2 * dv2 - 2.0 * uu2 * dg2
        return (a1, a2)
    zf = jnp.zeros((L,), jnp.float32)
    a1, a2 = lax.fori_loop(0, M // 2, body, (zf, zf))
    return a1 + a2


@functools.lru_cache(maxsize=1)
def _make_sw_kernel():
    mesh = plsc.VectorSubcoreMesh(core_axis_name="c", subcore_axis_name="s")
    return functools.partial(
        pl.kernel,
        mesh=mesh,
        compiler_params=pltpu.CompilerParams(needs_layout_passes=False),
        out_type=jax.ShapeDtypeStruct((BS * NPROJ * L,), jnp.float32),
        scratch_types=[
        pltpu.VMEM((N,), jnp.float32),
        pltpu.VMEM((N,), jnp.float32),
        pltpu.VMEM((N,), jnp.int32),
        pltpu.VMEM((N,), jnp.float32),
        pltpu.VMEM((N,), jnp.int32),
        pltpu.VMEM((N,), jnp.float32),
        pltpu.VMEM((N,), jnp.int32),
        pltpu.VMEM((N,), jnp.float32),
        pltpu.VMEM((N,), jnp.int32),
        pltpu.VMEM((N,), jnp.float32),
        pltpu.VMEM((N,), jnp.float32),
        pltpu.VMEM((N,), jnp.float32),
        pltpu.VMEM((N + L,), jnp.float32),
        pltpu.VMEM((N + L,), jnp.float32),
        pltpu.VMEM((N + L,), jnp.float32),
        pltpu.VMEM((N + L,), jnp.float32),
        pltpu.VMEM((NB * L,), jnp.int32),
        pltpu.VMEM((NB * L,), jnp.int32),
        pltpu.VMEM((NB * L,), jnp.int32),
        pltpu.VMEM((NB * L,), jnp.int32),
        pltpu.VMEM((N,), jnp.float32),
        pltpu.VMEM((N,), jnp.float32),
        pltpu.VMEM((TPW * L,), jnp.float32),
        ],
    )(_sw_body)


def _sw_body(up_hbm, vp_hbm, a_hbm, b_hbm, out_hbm, raw_u, raw_v,
             kA, pA, kB, pB, kC, pC, kD, pD,
             us, vs, u_pad, v_pad, c_pad, g_pad,
             hist_u, cnt_u, hist_v, cnt_v, a_loc, b_loc, res):
    s_idx = lax.axis_index("s")
    half = lax.axis_index("c")
    t0 = s_idx * NPROJ + half * TPW
    lanes = lax.iota(jnp.int32, L)

    pltpu.sync_copy(a_hbm.at[s_idx * 8], a_loc)
    pltpu.sync_copy(b_hbm.at[s_idx * 8], b_loc)

    def task(k, c):
        t = t0 + k
        pltpu.sync_copy(up_hbm.at[t], raw_u)
        pltpu.sync_copy(vp_hbm.at[t], raw_v)
        _build_keys2(raw_u, raw_v, a_loc, b_loc, kA, pA, kC, pC)
        _sort2(kA, pA, kB, pB, kC, pC, kD, pD,
               hist_u, cnt_u, hist_v, cnt_v, lanes)
        tot_u, tot_v = _finish2(kA, pA, kC, pC, us, vs, u_pad, v_pad)

        t_val = jnp.maximum(tot_u, tot_v)
        _clamp_last(u_pad, t_val, lanes)
        _clamp_last(v_pad, t_val, lanes)
        _build_c(vs, v_pad, c_pad)
        _search(u_pad, v_pad, c_pad, vs, g_pad)
        res[pl.ds(k * L, L)] = _accum(us, vs, u_pad, v_pad, g_pad)
        return c
    lax.fori_loop(0, TPW, task, 0)
    pltpu.sync_copy(res, out_hbm.at[pl.ds(t0 * L, TPW * L)])


def kernel(set1, set2, projections):
    xt = jnp.stack([set1.transpose(0, 2, 1), set2.transpose(0, 2, 1)])
    proj, w = _prep(xt, projections.T)
    up = proj[0].reshape(BS * NPROJ, N)
    vp = proj[1].reshape(BS * NPROJ, N)
    a2 = w[0].reshape(BS * 8, N)
    b2 = w[1].reshape(BS * 8, N)
    out = _make_sw_kernel()(up, vp, a2, b2)
    per_task = out.reshape(BS, NPROJ, L).sum(-1)
    return jnp.sum(jnp.mean(per_task, axis=1) ** 0.5)
